# baseline (speedup 1.0000x reference)
import functools

import jax
import jax.numpy as jnp
from jax import lax
from jax.experimental import pallas as pl
from jax.experimental.pallas import tpu as pltpu
from jax.experimental.pallas import tpu_sc as plsc

NUM_CORES = 2
NUM_SUBCORES = 16
NUM_WORKERS = NUM_CORES * NUM_SUBCORES
LANE = 128
D = 64
CB = 25
RBW = 4
GROUPS = CB * RBW
N_SLAB = GROUPS * 8
K = 6
NG = 8
NS = 4


def _build(batch, hist, vocab):
    assert batch == NUM_WORKERS * RBW * LANE and hist == CB * 8
    mesh = plsc.VectorSubcoreMesh(core_axis_name="c", subcore_axis_name="s")

    @functools.partial(
        pl.kernel,
        mesh=mesh,
        out_type=jax.ShapeDtypeStruct((hist, 8, LANE, 8 * LANE), jnp.float32),
        scratch_types=[
            pltpu.VMEM((2, 8, LANE), jnp.int32),
            pltpu.VMEM((NG, LANE, D), jnp.float32),
            pltpu.VMEM((NS * 8 * LANE * 8,), jnp.float32),
        ]
        + [pltpu.SemaphoreType.DMA] * (NG + NS + 2),
        compiler_params=pltpu.CompilerParams(
            use_tc_tiling_on_sc=False, needs_layout_passes=False
        ),
    )
    def gather_kernel(table_hbm, x4_hbm, out_hbm, idx_t, rows_v, slab_v, *sems):
        sem_g = sems[:NG]
        sem_s = sems[NG:NG + NS]
        sem_i = sems[NG + NS:]
        wid = lax.axis_index("s") * NUM_CORES + lax.axis_index("c")
        rb0 = wid * RBW

        iota = lax.iota(jnp.int32, 16)
        vd0 = [iota * 128 + d0 * 2048 for d0 in range(4)]

        def tile_coords(g):
            return g % CB, rb0 + g // CB

        def load_tile(g, slot):
            cb, rb = tile_coords(g)
            pltpu.make_async_copy(
                x4_hbm.at[cb, rb], idx_t.at[slot], sem_i[slot]
            ).start()

        def wait_tile(slot):
            pltpu.make_async_copy(
                x4_hbm.at[0, 0], idx_t.at[slot], sem_i[slot]
            ).wait()

        def gather_copy(tslot, cs):
            return pltpu.make_async_copy(
                table_hbm.at[idx_t.at[tslot, cs]], rows_v.at[cs], sem_g[cs]
            )

        def store_copies(g, cs):
            cb, rb = tile_coords(g)
            c = cb * 8 + cs
            s0 = (cs % NS) * 8192
            return [
                pltpu.make_async_copy(
                    slab_v.at[pl.ds(s0 + db * 1024, 1024)],
                    out_hbm.at[c, db, rb],
                    sem_s[cs % NS],
                )
                for db in range(8)
            ]

        def transpose(cs):
            src_rows = rows_v
            dst0 = (cs % NS) * 8192

            def body(l4, carry):
                l = l4 * 4
                for dl in range(4):
                    lane = l + dl
                    for d0 in range(4):
                        val = src_rows[cs, lane, pl.ds(d0 * 16, 16)]
                        plsc.store_scatter(
                            slab_v.at[pl.ds(dst0, 8192)],
                            [vd0[d0] + lane],
                            val,
                        )
                return carry

            lax.fori_loop(0, 32, body, 0)

        def slab_step(g, cs, g_par, first_group=False, last_group=False):
            if cs < 8 - K:
                gather_copy(g_par, cs + K).start()
            elif not last_group:
                if cs == 8 - K:
                    wait_tile(1 - g_par)
                gather_copy(1 - g_par, cs + K - 8).start()
            gather_copy(g_par, cs).wait()
            if not (first_group and cs < NS):
                for cp in store_copies(g, cs):
                    cp.wait()
            transpose(cs)
            for cp in store_copies(g, cs):
                cp.start()
            if cs == 7 and not last_group:
                load_tile(jnp.minimum(g + 2, GROUPS - 1), g_par)

        def group_body(g, g_par, **kw):
            for cs in range(8):
                slab_step(g, cs, g_par, **kw)

        load_tile(0, 0)
        wait_tile(0)
        for cs in range(K):
            gather_copy(0, cs).start()
        load_tile(1, 1)

        group_body(0, 0, first_group=True)

        def pair(p, carry):
            group_body(2 * p + 1, 1)
            group_body(2 * p + 2, 0)
            return carry

        lax.fori_loop(0, (GROUPS - 2) // 2, pair, 0)

        group_body(GROUPS - 1, 1, last_group=True)

        g_last = GROUPS - 1
        for cs in range(8 - NS, 8):
            for cp in store_copies(g_last, cs):
                cp.wait()
        wait_tile(0)

    return gather_kernel


def kernel(x, table):
    batch, hist = x.shape
    vocab, d_model = table.shape
    assert d_model == D
    x4 = (
        x.astype(jnp.int32)
        .T.reshape(CB, 8, LANE * NUM_WORKERS * RBW // LANE, LANE)
        .transpose(0, 2, 1, 3)
    )
    out4 = _build(batch, hist, vocab)(table, x4)
    out = (
        out4.reshape(hist, 8, batch // LANE, 8, LANE)
        .transpose(2, 4, 0, 1, 3)
        .reshape(batch, hist, D)
    )
    return out


# speedup vs baseline: 2.1485x; 2.1485x over previous
import functools

import jax
import jax.numpy as jnp
from jax import lax
from jax.experimental import pallas as pl
from jax.experimental.pallas import tpu as pltpu
from jax.experimental.pallas import tpu_sc as plsc

NUM_CORES = 2
NUM_SUBCORES = 16
NUM_WORKERS = NUM_CORES * NUM_SUBCORES
LANE = 128
D = 64
CB = 25
RBW = 4
GROUPS = CB * RBW
N_SLAB = GROUPS * 8
K = 6
NG = 8
NS = 4


def _build(batch, hist, vocab):
    assert batch == NUM_WORKERS * RBW * LANE and hist == CB * 8
    mesh = plsc.VectorSubcoreMesh(core_axis_name="c", subcore_axis_name="s")

    @functools.partial(
        pl.kernel,
        mesh=mesh,
        out_type=jax.ShapeDtypeStruct((hist, 8, LANE, 8, LANE), jnp.float32),
        scratch_types=[
            pltpu.VMEM((2, 8, LANE), jnp.int32),
            pltpu.VMEM((NG, LANE, D), jnp.float32),
            pltpu.VMEM((NS, 8, 8, LANE + 1), jnp.float32),
        ]
        + [pltpu.SemaphoreType.DMA] * (NG + NS + 2),
        compiler_params=pltpu.CompilerParams(
            use_tc_tiling_on_sc=False, needs_layout_passes=False
        ),
    )
    def gather_kernel(table_hbm, x4_hbm, out_hbm, idx_t, rows_v, slab_v, *sems):
        sem_g = sems[:NG]
        sem_s = sems[NG:NG + NS]
        sem_i = sems[NG + NS:]
        wid = lax.axis_index("s") * NUM_CORES + lax.axis_index("c")
        rb0 = wid * RBW

        iota = lax.iota(jnp.int32, 16)
        db_vecs = [d0 * 2 + iota // 8 for d0 in range(4)]
        ds_vec = iota % 8

        def tile_coords(g):
            return g % CB, rb0 + g // CB

        def load_tile(g, slot):
            cb, rb = tile_coords(g)
            pltpu.make_async_copy(
                x4_hbm.at[cb, rb], idx_t.at[slot], sem_i[slot]
            ).start()

        def wait_tile(slot):
            pltpu.make_async_copy(
                x4_hbm.at[0, 0], idx_t.at[slot], sem_i[slot]
            ).wait()

        def gather_copy(tslot, cs):
            return pltpu.make_async_copy(
                table_hbm.at[idx_t.at[tslot, cs]], rows_v.at[cs], sem_g[cs]
            )

        def store_copies(g, cs):
            cb, rb = tile_coords(g)
            c = cb * 8 + cs
            slot = cs % NS
            return [
                pltpu.make_async_copy(
                    slab_v.at[slot, :, :, pl.ds(0, LANE)],
                    out_hbm.at[c, :, rb],
                    sem_s[slot],
                )
            ]

        def transpose(cs):
            slot = cs % NS

            def body(l4, carry):
                l = l4 * 4
                for dl in range(4):
                    lane = l + dl
                    lane_vec = jnp.full((16,), 0, jnp.int32) + lane
                    for d0 in range(4):
                        val = rows_v[cs, lane, pl.ds(d0 * 16, 16)]
                        plsc.store_scatter(
                            slab_v.at[slot],
                            [db_vecs[d0], ds_vec, lane_vec],
                            val,
                        )
                return carry

            lax.fori_loop(0, 32, body, 0)

        def slab_step(g, cs, g_par, first_group=False, last_group=False):
            if cs < 8 - K:
                gather_copy(g_par, cs + K).start()
            elif not last_group:
                if cs == 8 - K:
                    wait_tile(1 - g_par)
                gather_copy(1 - g_par, cs + K - 8).start()
            gather_copy(g_par, cs).wait()
            if not (first_group and cs < NS):
                for cp in store_copies(g, cs):
                    cp.wait()
            transpose(cs)
            for cp in store_copies(g, cs):
                cp.start()
            if cs == 7 and not last_group:
                load_tile(jnp.minimum(g + 2, GROUPS - 1), g_par)

        def group_body(g, g_par, **kw):
            for cs in range(8):
                slab_step(g, cs, g_par, **kw)

        load_tile(0, 0)
        wait_tile(0)
        for cs in range(K):
            gather_copy(0, cs).start()
        load_tile(1, 1)

        group_body(0, 0, first_group=True)

        def pair(p, carry):
            group_body(2 * p + 1, 1)
            group_body(2 * p + 2, 0)
            return carry

        lax.fori_loop(0, (GROUPS - 2) // 2, pair, 0)

        group_body(GROUPS - 1, 1, last_group=True)

        g_last = GROUPS - 1
        for cs in range(8 - NS, 8):
            for cp in store_copies(g_last, cs):
                cp.wait()
        wait_tile(0)

    return gather_kernel


def kernel(x, table):
    batch, hist = x.shape
    vocab, d_model = table.shape
    assert d_model == D
    x4 = (
        x.astype(jnp.int32)
        .T.reshape(CB, 8, LANE * NUM_WORKERS * RBW // LANE, LANE)
        .transpose(0, 2, 1, 3)
    )
    out5 = _build(batch, hist, vocab)(table, x4)
    return out5.transpose(2, 4, 0, 1, 3).reshape(batch, hist, D)


# speedup vs baseline: 2.1580x; 1.0044x over previous
import functools

import jax
import jax.numpy as jnp
from jax import lax
from jax.experimental import pallas as pl
from jax.experimental.pallas import tpu as pltpu
from jax.experimental.pallas import tpu_sc as plsc

NUM_CORES = 2
NUM_SUBCORES = 16
NUM_WORKERS = NUM_CORES * NUM_SUBCORES
LANE = 128
D = 64
CB = 25
RBW = 4
GROUPS = CB * RBW
N_SLAB = GROUPS * 8
K = 6
NG = 8
NS = 4


def _build(batch, hist, vocab):
    assert batch == NUM_WORKERS * RBW * LANE and hist == CB * 8
    mesh = plsc.VectorSubcoreMesh(core_axis_name="c", subcore_axis_name="s")

    @functools.partial(
        pl.kernel,
        mesh=mesh,
        out_type=jax.ShapeDtypeStruct((hist, 8, LANE, 8, LANE), jnp.float32),
        scratch_types=[
            pltpu.VMEM((2, 8, LANE), jnp.int32),
            pltpu.VMEM((NG, LANE, D), jnp.float32),
            pltpu.VMEM((NS, 8, 8, LANE + 1), jnp.float32),
        ]
        + [pltpu.SemaphoreType.DMA] * (NG + NS + 2),
        compiler_params=pltpu.CompilerParams(
            use_tc_tiling_on_sc=False, needs_layout_passes=False
        ),
    )
    def gather_kernel(table_hbm, x4_hbm, out_hbm, idx_t, rows_v, slab_v, *sems):
        sem_g = sems[:NG]
        sem_s = sems[NG:NG + NS]
        sem_i = sems[NG + NS:]
        wid = lax.axis_index("s") * NUM_CORES + lax.axis_index("c")
        rb0 = wid * RBW

        iota = lax.iota(jnp.int32, 16)
        db_vecs = [d0 * 2 + iota // 8 for d0 in range(4)]
        ds_vec = iota % 8

        def tile_coords(g):
            return g % CB, rb0 + g // CB

        def load_tile(g, slot):
            cb, rb = tile_coords(g)
            pltpu.make_async_copy(
                x4_hbm.at[cb, rb], idx_t.at[slot], sem_i[slot]
            ).start()

        def wait_tile(slot):
            pltpu.make_async_copy(
                x4_hbm.at[0, 0], idx_t.at[slot], sem_i[slot]
            ).wait()

        def gather_copy(tslot, cs):
            return pltpu.make_async_copy(
                table_hbm.at[idx_t.at[tslot, cs]], rows_v.at[cs], sem_g[cs]
            )

        def store_copies(g, cs):
            cb, rb = tile_coords(g)
            c = cb * 8 + cs
            slot = cs % NS
            return [
                pltpu.make_async_copy(
                    slab_v.at[slot, :, :, pl.ds(0, LANE)],
                    out_hbm.at[c, :, rb],
                    sem_s[slot],
                )
            ]

        def transpose(cs):
            slot = cs % NS

            def body(l8, carry):
                l = l8 * 8
                base_vec = jnp.full((16,), 0, jnp.int32) + l
                for dl in range(8):
                    lane = l + dl
                    lane_vec = base_vec + dl
                    for d0 in range(4):
                        val = rows_v[cs, lane, pl.ds(d0 * 16, 16)]
                        plsc.store_scatter(
                            slab_v.at[slot],
                            [db_vecs[d0], ds_vec, lane_vec],
                            val,
                        )
                return carry

            lax.fori_loop(0, 16, body, 0)

        def slab_step(g, cs, g_par, first_group=False, last_group=False):
            if cs < 8 - K:
                gather_copy(g_par, cs + K).start()
            elif not last_group:
                if cs == 8 - K:
                    wait_tile(1 - g_par)
                gather_copy(1 - g_par, cs + K - 8).start()
            gather_copy(g_par, cs).wait()
            if not (first_group and cs < NS):
                for cp in store_copies(g, cs):
                    cp.wait()
            transpose(cs)
            for cp in store_copies(g, cs):
                cp.start()
            if cs == 7 and not last_group:
                load_tile(jnp.minimum(g + 2, GROUPS - 1), g_par)

        def group_body(g, g_par, **kw):
            for cs in range(8):
                slab_step(g, cs, g_par, **kw)

        load_tile(0, 0)
        wait_tile(0)
        for cs in range(K):
            gather_copy(0, cs).start()
        load_tile(1, 1)

        group_body(0, 0, first_group=True)

        def pair(p, carry):
            group_body(2 * p + 1, 1)
            group_body(2 * p + 2, 0)
            return carry

        lax.fori_loop(0, (GROUPS - 2) // 2, pair, 0)

        group_body(GROUPS - 1, 1, last_group=True)

        g_last = GROUPS - 1
        for cs in range(8 - NS, 8):
            for cp in store_copies(g_last, cs):
                cp.wait()
        wait_tile(0)

    return gather_kernel


def kernel(x, table):
    batch, hist = x.shape
    vocab, d_model = table.shape
    assert d_model == D
    x4 = (
        x.astype(jnp.int32)
        .T.reshape(CB, 8, LANE * NUM_WORKERS * RBW // LANE, LANE)
        .transpose(0, 2, 1, 3)
    )
    out5 = _build(batch, hist, vocab)(table, x4)
    return out5.transpose(2, 4, 0, 1, 3).reshape(batch, hist, D)


# speedup vs baseline: 2.2092x; 1.0237x over previous
import functools

import jax
import jax.numpy as jnp
from jax import lax
from jax.experimental import pallas as pl
from jax.experimental.pallas import tpu as pltpu
from jax.experimental.pallas import tpu_sc as plsc

NUM_CORES = 2
NUM_SUBCORES = 16
NUM_WORKERS = NUM_CORES * NUM_SUBCORES
LANE = 128
D = 64
CB = 25
RBW = 4
GROUPS = CB * RBW
N_SLAB = GROUPS * 8
K = 6
NG = 8
NS = 4


def _build(batch, hist, vocab):
    assert batch == NUM_WORKERS * RBW * LANE and hist == CB * 8
    mesh = plsc.VectorSubcoreMesh(core_axis_name="c", subcore_axis_name="s")

    @functools.partial(
        pl.kernel,
        mesh=mesh,
        out_type=jax.ShapeDtypeStruct((hist, 8, LANE, 8, LANE), jnp.float32),
        scratch_types=[
            pltpu.VMEM((2, 8, LANE), jnp.int32),
            pltpu.VMEM((NG, LANE, D), jnp.float32),
            pltpu.VMEM((NS, 8, 8, LANE + 1), jnp.float32),
        ]
        + [pltpu.SemaphoreType.DMA] * (NG + NS + 2),
        compiler_params=pltpu.CompilerParams(
            use_tc_tiling_on_sc=False, needs_layout_passes=False
        ),
    )
    def gather_kernel(table_hbm, x4_hbm, out_hbm, idx_t, rows_v, slab_v, *sems):
        sem_g = sems[:NG]
        sem_s = sems[NG:NG + NS]
        sem_i = sems[NG + NS:]
        wid = lax.axis_index("s") * NUM_CORES + lax.axis_index("c")
        rb0 = wid * RBW

        iota = lax.iota(jnp.int32, 16)
        db_vecs = [d0 * 2 + iota // 8 for d0 in range(4)]
        ds_vec = iota % 8

        def tile_coords(g):
            return g % CB, rb0 + g // CB

        def load_tile(g, slot):
            cb, rb = tile_coords(g)
            pltpu.make_async_copy(
                x4_hbm.at[cb, rb], idx_t.at[slot], sem_i[slot]
            ).start()

        def wait_tile(slot):
            pltpu.make_async_copy(
                x4_hbm.at[0, 0], idx_t.at[slot], sem_i[slot]
            ).wait()

        def gather_copy(tslot, cs):
            return pltpu.make_async_copy(
                table_hbm.at[idx_t.at[tslot, cs]], rows_v.at[cs], sem_g[cs]
            )

        def store_copies(g, cs):
            cb, rb = tile_coords(g)
            c = cb * 8 + cs
            slot = cs % NS
            return [
                pltpu.make_async_copy(
                    slab_v.at[slot, :, :, pl.ds(0, LANE)],
                    out_hbm.at[c, :, rb],
                    sem_s[slot],
                )
            ]

        def transpose(cs):
            slot = cs % NS

            def body(l8, carry):
                l = l8 * 8
                base_vec = jnp.full((16,), 0, jnp.int32) + l
                for dl in range(8):
                    lane = l + dl
                    lane_vec = base_vec + dl
                    for d0 in range(4):
                        val = rows_v[cs, lane, pl.ds(d0 * 16, 16)]
                        plsc.store_scatter(
                            slab_v.at[slot],
                            [db_vecs[d0], ds_vec, lane_vec],
                            val,
                        )
                return carry

            lax.fori_loop(0, 16, body, 0)

        def slab_step(g, cs, g_par, first_group=False, last_group=False):
            if cs < 8 - K:
                gather_copy(g_par, cs + K).start()
            elif not last_group:
                if cs == 8 - K:
                    wait_tile(1 - g_par)
                gather_copy(1 - g_par, cs + K - 8).start()
            gather_copy(g_par, cs).wait()
            if not (first_group and cs < NS):
                for cp in store_copies(g, cs):
                    cp.wait()
            transpose(cs)
            for cp in store_copies(g, cs):
                cp.start()
            if cs == 7 and not last_group:
                load_tile(jnp.minimum(g + 2, GROUPS - 1), g_par)

        def group_body(g, g_par, **kw):
            for cs in range(8):
                slab_step(g, cs, g_par, **kw)

        load_tile(0, 0)
        wait_tile(0)
        for cs in range(K):
            gather_copy(0, cs).start()
        load_tile(1, 1)

        group_body(0, 0, first_group=True)

        def pair(p, carry):
            group_body(2 * p + 1, 1)
            group_body(2 * p + 2, 0)
            return carry

        lax.fori_loop(0, (GROUPS - 2) // 2, pair, 0)

        group_body(GROUPS - 1, 1, last_group=True)

        g_last = GROUPS - 1
        for cs in range(8 - NS, 8):
            for cp in store_copies(g_last, cs):
                cp.wait()
        wait_tile(0)

    return gather_kernel


def kernel(x, table):
    batch, hist = x.shape
    vocab, d_model = table.shape
    assert d_model == D
    table2 = jnp.pad(table, ((0, 0), (0, D))).reshape(2 * vocab, D)
    x4 = (
        (x.astype(jnp.int32) * 2)
        .T.reshape(CB, 8, LANE * NUM_WORKERS * RBW // LANE, LANE)
        .transpose(0, 2, 1, 3)
    )
    out5 = _build(batch, hist, 2 * vocab)(table2, x4)
    return out5.transpose(2, 4, 0, 1, 3).reshape(batch, hist, D)


# speedup vs baseline: 2.2242x; 1.0068x over previous
import functools

import jax
import jax.numpy as jnp
from jax import lax
from jax.experimental import pallas as pl
from jax.experimental.pallas import tpu as pltpu
from jax.experimental.pallas import tpu_sc as plsc

NUM_CORES = 2
NUM_SUBCORES = 16
NUM_WORKERS = NUM_CORES * NUM_SUBCORES
LANE = 128
D = 64
CB = 25
RBW = 4
GROUPS = CB * RBW
K = 6
NG = 8
NS = 4


def _build(batch, hist, vocab):
    assert batch == NUM_WORKERS * RBW * LANE and hist == CB * 8
    mesh = plsc.VectorSubcoreMesh(core_axis_name="c", subcore_axis_name="s")

    @functools.partial(
        pl.kernel,
        mesh=mesh,
        out_type=jax.ShapeDtypeStruct((hist, 8, LANE, 8, LANE), jnp.float32),
        scratch_types=[
            pltpu.VMEM((2, 8, LANE), jnp.int32),
            pltpu.VMEM((NG, LANE, D), jnp.float32),
            pltpu.VMEM((NS, 8, 8, LANE + 1), jnp.float32),
        ]
        + [pltpu.SemaphoreType.DMA] * (NG + NS + 2),
        compiler_params=pltpu.CompilerParams(
            use_tc_tiling_on_sc=False, needs_layout_passes=False
        ),
    )
    def gather_kernel(table_hbm, x4_hbm, out_hbm, idx_t, rows_v, slab_v, *sems):
        sem_g = sems[:NG]
        sem_s = sems[NG:NG + NS]
        sem_i = sems[NG + NS:]
        wid = lax.axis_index("s") * NUM_CORES + lax.axis_index("c")
        rb0 = wid * RBW

        iota = lax.iota(jnp.int32, 16)
        db_vecs = [d0 * 2 + iota // 8 for d0 in range(4)]
        ds_vec = iota % 8

        def tile_coords(g):
            return g % CB, rb0 + g // CB

        def load_tile(g, slot):
            cb, rb = tile_coords(g)
            pltpu.make_async_copy(
                x4_hbm.at[cb, rb], idx_t.at[slot], sem_i[slot]
            ).start()

        def wait_tile(slot):
            pltpu.make_async_copy(
                x4_hbm.at[0, 0], idx_t.at[slot], sem_i[slot]
            ).wait()

        def gather_copy(tslot, cs):
            return pltpu.make_async_copy(
                table_hbm.at[idx_t.at[tslot, cs]], rows_v.at[cs], sem_g[cs]
            )

        def store_copies(g, cs):
            cb, rb = tile_coords(g)
            c = cb * 8 + cs
            slot = cs % NS
            return [
                pltpu.make_async_copy(
                    slab_v.at[slot, :, :, pl.ds(0, LANE)],
                    out_hbm.at[c, :, rb],
                    sem_s[slot],
                )
            ]

        def transpose(cs):
            slot = cs % NS

            def body(l8, carry):
                l = l8 * 8
                base_vec = jnp.full((16,), 0, jnp.int32) + l
                for dl in range(8):
                    lane = l + dl
                    lane_vec = base_vec + dl
                    for d0 in range(4):
                        val = rows_v[cs, lane, pl.ds(d0 * 16, 16)]
                        plsc.store_scatter(
                            slab_v.at[slot],
                            [db_vecs[d0], ds_vec, lane_vec],
                            val,
                        )
                return carry

            lax.fori_loop(0, 16, body, 0)

        def slab_step(g, cs, g_par, first_group=False, last_group=False):
            if cs < 8 - K:
                gather_copy(g_par, cs + K).start()
            elif not last_group:
                if cs == 8 - K:
                    wait_tile(1 - g_par)
                gather_copy(1 - g_par, cs + K - 8).start()
            gather_copy(g_par, cs).wait()
            if not (first_group and cs < NS):
                for cp in store_copies(g, cs):
                    cp.wait()
            transpose(cs)
            for cp in store_copies(g, cs):
                cp.start()
            if cs == 7 and not last_group:
                load_tile(jnp.minimum(g + 2, GROUPS - 1), g_par)

        def group_body(g, g_par, **kw):
            for cs in range(8):
                slab_step(g, cs, g_par, **kw)

        load_tile(0, 0)
        wait_tile(0)
        for cs in range(K):
            gather_copy(0, cs).start()
        load_tile(1, 1)

        group_body(0, 0, first_group=True)

        def pair(p, carry):
            group_body(2 * p + 1, 1)
            group_body(2 * p + 2, 0)
            return carry

        lax.fori_loop(0, (GROUPS - 2) // 2, pair, 0)

        group_body(GROUPS - 1, 1, last_group=True)

        g_last = GROUPS - 1
        for cs in range(8 - NS, 8):
            for cp in store_copies(g_last, cs):
                cp.wait()
        wait_tile(0)

    return gather_kernel


def kernel(x, table):
    batch, hist = x.shape
    vocab, d_model = table.shape
    assert d_model == D
    table2 = jnp.pad(table, ((0, 0), (0, D))).reshape(2 * vocab, D)
    x4 = (
        (x.astype(jnp.int32) * 2)
        .T.reshape(CB, 8, LANE * NUM_WORKERS * RBW // LANE, LANE)
        .transpose(0, 2, 1, 3)
    )
    out5 = _build(batch, hist, 2 * vocab)(table2, x4)
    return out5.transpose(2, 4, 0, 1, 3).reshape(batch, hist, D)
